# baseline (speedup 1.0000x reference)
import jax
from jax.experimental import pallas as pl


_R_BLK = 2048


def _add_kernel(in_ref, emb_ref, out_ref):
    i = pl.program_id(0)
    seq_blocks = emb_ref.shape[0] // _R_BLK
    e0 = pl.multiple_of((i % seq_blocks) * _R_BLK, _R_BLK)
    out_ref[...] = in_ref[...] + emb_ref[pl.ds(e0, _R_BLK), :]


def kernel(inputs, embeddings):
    batch, seq_len, dim = inputs.shape
    pos = embeddings[:seq_len]
    flat = inputs.reshape(batch * seq_len, dim)
    grid = (batch * seq_len // _R_BLK,)
    out = pl.pallas_call(
        _add_kernel,
        grid=grid,
        in_specs=[
            pl.BlockSpec((_R_BLK, dim), lambda i: (i, 0)),
            pl.BlockSpec((seq_len, dim), lambda i: (0, 0)),
        ],
        out_specs=pl.BlockSpec((_R_BLK, dim), lambda i: (i, 0)),
        out_shape=jax.ShapeDtypeStruct((batch * seq_len, dim), inputs.dtype),
    )(flat, pos)
    return out.reshape(batch, seq_len, dim)
